# baseline (speedup 1.0000x reference)
import jax
import jax.numpy as jnp
from jax.experimental import pallas as pl

_K = 6
_TILE = 128
_MASKVAL = 1e30
_TAKEN = 1e32
_BIGIDX = 2**30


def _knn_kernel(feat_ref, featT_ref, brow_ref, bcol_ref, b_ref, bg_ref):
    i = pl.program_id(0)
    feat = feat_ref[...]
    ft = featT_ref[...]
    sqc = jnp.sum(ft * ft, axis=0, keepdims=True)
    d = sqc - 2.0 * jnp.dot(feat, ft, preferred_element_type=jnp.float32)
    brow = brow_ref[...]
    bcol = bcol_ref[...]
    bfull = b_ref[...]
    colv = jax.lax.broadcasted_iota(jnp.int32, (1, d.shape[1]), 1)
    rowv = i * _TILE + jax.lax.broadcasted_iota(jnp.int32, (_TILE, 1), 0)
    d = jnp.where((brow != bcol) | (rowv == colv), _MASKVAL, d)
    outs = []
    for _ in range(_K):
        m = jnp.min(d, axis=1, keepdims=True)
        a = jnp.min(jnp.where(d == m, colv, _BIGIDX), axis=1).astype(jnp.int32)
        sel = colv == a[:, None]
        outs.append(jnp.dot(sel.astype(jnp.float32), bfull,
                            preferred_element_type=jnp.float32))
        d = jnp.where(sel, _TAKEN, d)
    bg_ref[...] = jnp.concatenate(outs, axis=1)


def _knn_gather(feat, batch, b):
    n, c = feat.shape
    cb = b.shape[1]
    npad = ((n + _TILE - 1) // _TILE) * _TILE
    dp = ((c + 7) // 8) * 8
    fp = jnp.zeros((npad, dp), jnp.float32).at[:n, :c].set(feat)
    bp = jnp.full((npad,), -1, jnp.int32).at[:n].set(batch)
    bpad = jnp.zeros((npad, cb), jnp.float32).at[:n].set(b)
    bg = pl.pallas_call(
        _knn_kernel,
        grid=(npad // _TILE,),
        in_specs=[
            pl.BlockSpec((_TILE, dp), lambda i: (i, 0)),
            pl.BlockSpec((dp, npad), lambda i: (0, 0)),
            pl.BlockSpec((_TILE, 1), lambda i: (i, 0)),
            pl.BlockSpec((1, npad), lambda i: (0, 0)),
            pl.BlockSpec((npad, cb), lambda i: (0, 0)),
        ],
        out_specs=pl.BlockSpec((_TILE, _K * cb), lambda i: (i, 0)),
        out_shape=jax.ShapeDtypeStruct((npad, _K * cb), jnp.float32),
    )(fp, fp.T, bp[:, None], bp[None, :], bpad)
    return bg[:n].reshape(n, _K, cb)


def _agg_kernel(adj_ref, g_ref, b_ref, out_ref):
    out_ref[...] = (
        jnp.dot(adj_ref[...], g_ref[...], preferred_element_type=jnp.float32)
        + b_ref[...]
    )


def _agg(adj, g, b):
    npad, c = g.shape
    return pl.pallas_call(
        _agg_kernel,
        grid=(npad // _TILE,),
        in_specs=[
            pl.BlockSpec((_TILE, npad), lambda i: (i, 0)),
            pl.BlockSpec((npad, c), lambda i: (0, 0)),
            pl.BlockSpec((1, c), lambda i: (0, 0)),
        ],
        out_specs=pl.BlockSpec((_TILE, c), lambda i: (i, 0)),
        out_shape=jax.ShapeDtypeStruct((npad, c), jnp.float32),
    )(adj, g, b[None, :])


def _leaky(v):
    return jnp.where(v >= 0, v, 0.1 * v)


def _bnorm(v, g, b):
    m = jnp.mean(v, axis=0)
    var = jnp.var(v, axis=0)
    return (v - m) / jnp.sqrt(var + 1e-5) * g + b


def _edge_conv(feat, batch, p):
    c = feat.shape[1]
    n = feat.shape[0]
    a = feat @ (p["W1"][:c] - p["W1"][c:]) + p["b1"]
    b = feat @ p["W1"][c:]
    bg = _knn_gather(feat, batch, b)
    h = _leaky(a[:, None, :] + bg).reshape(n * _K, -1)
    h = _bnorm(h, p["g1"], p["be1"])
    h = _leaky(h @ p["W2"] + p["b2"])
    h = _bnorm(h, p["g2"], p["be2"])
    s = h.reshape(n, _K, -1).sum(axis=1)
    return s @ p["W3"] + _K * p["b3"]


def kernel(x, pos, edge_index, batch, ec_params, gcn_params):
    h = _edge_conv(pos, batch, ec_params[0])
    append = h
    for p in ec_params[1:3]:
        h = _edge_conv(h, batch, p)
    h = _edge_conv(h, batch, ec_params[3])
    err = _edge_conv(jnp.concatenate([append, h], axis=1), batch, ec_params[4])

    n = x.shape[0]
    npad = ((n + _TILE - 1) // _TILE) * _TILE
    loops = jnp.arange(n, dtype=edge_index.dtype)
    src = jnp.concatenate([edge_index[0], loops])
    dst = jnp.concatenate([edge_index[1], loops])
    deg = jnp.zeros((n,), jnp.float32).at[dst].add(1.0)
    dinv = jnp.where(deg > 0, 1.0 / jnp.sqrt(deg), 0.0)
    norm = dinv[src] * dinv[dst]
    adj = jnp.zeros((npad, npad), jnp.float32).at[dst, src].add(norm)

    def gcn(feat, p):
        g = jnp.zeros((npad, p["W"].shape[1]), jnp.float32).at[:n].set(feat @ p["W"])
        return _agg(adj, g, p["b"])[:n]

    u = _leaky(gcn(jnp.concatenate([x, err], axis=1), gcn_params[0]))
    for p in gcn_params[1:4]:
        u = _leaky(gcn(u, p))
    u = gcn(u, gcn_params[4])
    return u


# speedup vs baseline: 1.0646x; 1.0646x over previous
import jax
import jax.numpy as jnp
from jax.experimental import pallas as pl

_K = 6
_TILE = 128
_MASKVAL = 1e30
_TAKEN = 1e32
_BIGIDX = 2**30


def _knn_kernel(feat_ref, featT_ref, brow_ref, bcol_ref, idx_ref):
    i = pl.program_id(0)
    feat = feat_ref[...]
    ft = featT_ref[...]
    sqc = jnp.sum(ft * ft, axis=0, keepdims=True)
    d = sqc - 2.0 * jnp.dot(feat, ft, preferred_element_type=jnp.float32)
    brow = brow_ref[...]
    bcol = bcol_ref[...]
    colv = jax.lax.broadcasted_iota(jnp.int32, (1, d.shape[1]), 1)
    rowv = i * _TILE + jax.lax.broadcasted_iota(jnp.int32, (_TILE, 1), 0)
    d = jnp.where((brow != bcol) | (rowv == colv), _MASKVAL, d)
    cols = []
    for _ in range(_K):
        m = jnp.min(d, axis=1, keepdims=True)
        a = jnp.min(jnp.where(d == m, colv, _BIGIDX), axis=1).astype(jnp.int32)
        cols.append(a)
        d = jnp.where(colv == a[:, None], _TAKEN, d)
    idx_ref[...] = jnp.stack(cols, axis=1)


def _knn(feat, batch):
    n, c = feat.shape
    npad = ((n + _TILE - 1) // _TILE) * _TILE
    dp = ((c + 7) // 8) * 8
    fp = jnp.zeros((npad, dp), jnp.float32).at[:n, :c].set(feat)
    bp = jnp.full((npad,), -1, jnp.int32).at[:n].set(batch)
    idx = pl.pallas_call(
        _knn_kernel,
        grid=(npad // _TILE,),
        in_specs=[
            pl.BlockSpec((_TILE, dp), lambda i: (i, 0)),
            pl.BlockSpec((dp, npad), lambda i: (0, 0)),
            pl.BlockSpec((_TILE, 1), lambda i: (i, 0)),
            pl.BlockSpec((1, npad), lambda i: (0, 0)),
        ],
        out_specs=pl.BlockSpec((_TILE, _K), lambda i: (i, 0)),
        out_shape=jax.ShapeDtypeStruct((npad, _K), jnp.int32),
    )(fp, fp.T, bp[:, None], bp[None, :])
    return idx[:n]


def _agg_kernel(adj_ref, g_ref, b_ref, out_ref):
    out_ref[...] = (
        jnp.dot(adj_ref[...], g_ref[...], preferred_element_type=jnp.float32)
        + b_ref[...]
    )


def _agg(adj, g, b):
    npad, c = g.shape
    return pl.pallas_call(
        _agg_kernel,
        grid=(npad // _TILE,),
        in_specs=[
            pl.BlockSpec((_TILE, npad), lambda i: (i, 0)),
            pl.BlockSpec((npad, c), lambda i: (0, 0)),
            pl.BlockSpec((1, c), lambda i: (0, 0)),
        ],
        out_specs=pl.BlockSpec((_TILE, c), lambda i: (i, 0)),
        out_shape=jax.ShapeDtypeStruct((npad, c), jnp.float32),
    )(adj, g, b[None, :])


def _leaky(v):
    return jnp.where(v >= 0, v, 0.1 * v)


def _bnorm(v, g, b):
    m = jnp.mean(v, axis=0)
    var = jnp.var(v, axis=0)
    return (v - m) / jnp.sqrt(var + 1e-5) * g + b


def _edge_conv(feat, batch, p):
    idx = _knn(feat, batch)
    c = feat.shape[1]
    n = feat.shape[0]
    a = feat @ (p["W1"][:c] - p["W1"][c:]) + p["b1"]
    b = feat @ p["W1"][c:]
    h = _leaky(a[:, None, :] + b[idx]).reshape(n * _K, -1)
    h = _bnorm(h, p["g1"], p["be1"])
    h = _leaky(h @ p["W2"] + p["b2"])
    h = _bnorm(h, p["g2"], p["be2"])
    s = h.reshape(n, _K, -1).sum(axis=1)
    return s @ p["W3"] + _K * p["b3"]


def kernel(x, pos, edge_index, batch, ec_params, gcn_params):
    h = _edge_conv(pos, batch, ec_params[0])
    append = h
    for p in ec_params[1:3]:
        h = _edge_conv(h, batch, p)
    h = _edge_conv(h, batch, ec_params[3])
    err = _edge_conv(jnp.concatenate([append, h], axis=1), batch, ec_params[4])

    n = x.shape[0]
    npad = ((n + _TILE - 1) // _TILE) * _TILE
    loops = jnp.arange(n, dtype=edge_index.dtype)
    src = jnp.concatenate([edge_index[0], loops])
    dst = jnp.concatenate([edge_index[1], loops])
    deg = jnp.zeros((n,), jnp.float32).at[dst].add(1.0)
    dinv = jnp.where(deg > 0, 1.0 / jnp.sqrt(deg), 0.0)
    norm = dinv[src] * dinv[dst]
    adj = jnp.zeros((npad, npad), jnp.float32).at[dst, src].add(norm)

    def gcn(feat, p):
        g = jnp.zeros((npad, p["W"].shape[1]), jnp.float32).at[:n].set(feat @ p["W"])
        return _agg(adj, g, p["b"])[:n]

    u = _leaky(gcn(jnp.concatenate([x, err], axis=1), gcn_params[0]))
    for p in gcn_params[1:4]:
        u = _leaky(gcn(u, p))
    u = gcn(u, gcn_params[4])
    return u
